# Initial kernel scaffold; baseline (speedup 1.0000x reference)
import jax
import jax.numpy as jnp
from jax.experimental import pallas as pl


def kernel(x, Wg, bg, We, be):
    raise NotImplementedError("write your pallas kernel here")


# speedup vs baseline: 1.7552x; 1.7552x over previous
import functools

import jax
import jax.numpy as jnp
from jax.experimental import pallas as pl
from jax.experimental.pallas import tpu as pltpu

N = 8192
D = 4096
E = 8
H = 512
K = 2
BM = 256
M_BLOCKS = N // BM


def _moe_kernel(x_ref, wg_ref, bg_ref, we_hbm, be_ref,
                out_ref, loss_ref,
                we_vmem, psum_ref, copy_sem):
    m = pl.program_id(0)

    @pl.when(m == 0)
    def _load_weights():
        cp = pltpu.make_async_copy(we_hbm, we_vmem, copy_sem)
        cp.start()
        cp.wait()

    x_blk = x_ref[...]

    logits = jax.lax.dot_general(
        x_blk.astype(jnp.bfloat16), wg_ref[...].astype(jnp.bfloat16),
        dimension_numbers=(((1,), (0,)), ((), ())),
        preferred_element_type=jnp.float32) + bg_ref[...]
    mx = jnp.max(logits, axis=1, keepdims=True)
    ex = jnp.exp(logits - mx)
    p = ex / jnp.sum(ex, axis=1, keepdims=True)

    idx = jax.lax.broadcasted_iota(jnp.int32, (BM, E), 1)
    v1 = jnp.max(p, axis=1, keepdims=True)
    i1 = jnp.min(jnp.where(p == v1, idx, E), axis=1, keepdims=True)
    p_m = jnp.where(idx == i1, -1.0, p)
    v2 = jnp.max(p_m, axis=1, keepdims=True)
    i2 = jnp.min(jnp.where(p_m == v2, idx, E), axis=1, keepdims=True)
    w = jnp.where(idx == i1, v1, 0.0) + jnp.where(idx == i2, v2, 0.0)

    @pl.when(m == 0)
    def _init_psum():
        psum_ref[...] = jnp.zeros_like(psum_ref)

    psum_ref[...] += jnp.sum(p, axis=0, keepdims=True)

    xb = x_blk.astype(jnp.bfloat16)
    acc = jnp.zeros((BM, H), dtype=jnp.float32)
    for e in range(E):
        mm = jax.lax.dot_general(
            xb, we_vmem[e],
            dimension_numbers=(((1,), (0,)), ((), ())),
            preferred_element_type=jnp.float32)
        acc = acc + w[:, e:e + 1] * (mm + be_ref[e][None, :])

    out_ref[...] = jnp.tile(acc, (1, E))

    @pl.when(m == M_BLOCKS - 1)
    def _loss():
        mean_p = psum_ref[...] / N
        loss_ref[...] = jnp.sum((mean_p - 1.0 / E) ** 2, keepdims=True)


@jax.jit
def kernel(x, Wg, bg, We, be):
    we_bf = We.astype(jnp.bfloat16)
    bg2 = bg.reshape(1, E)
    out, loss = pl.pallas_call(
        _moe_kernel,
        grid=(M_BLOCKS,),
        in_specs=[
            pl.BlockSpec((BM, D), lambda m: (m, 0)),
            pl.BlockSpec((D, E), lambda m: (0, 0)),
            pl.BlockSpec((1, E), lambda m: (0, 0)),
            pl.BlockSpec(memory_space=pl.ANY),
            pl.BlockSpec((E, H), lambda m: (0, 0)),
        ],
        out_specs=[
            pl.BlockSpec((BM, E * H), lambda m: (m, 0)),
            pl.BlockSpec((1, 1), lambda m: (0, 0)),
        ],
        out_shape=[
            jax.ShapeDtypeStruct((N, E * H), jnp.float32),
            jax.ShapeDtypeStruct((1, 1), jnp.float32),
        ],
        scratch_shapes=[
            pltpu.VMEM((E, D, H), jnp.bfloat16),
            pltpu.VMEM((1, E), jnp.float32),
            pltpu.SemaphoreType.DMA,
        ],
        compiler_params=pltpu.CompilerParams(
            dimension_semantics=("arbitrary",),
        ),
    )(x, Wg, bg2, we_bf, be)
    return out, loss.reshape(())
